# baseline (speedup 1.0000x reference)
import functools

import jax
import jax.numpy as jnp
from jax import lax
from jax.experimental import pallas as pl
from jax.experimental.pallas import tpu as pltpu
from jax.experimental.pallas import tpu_sc as plsc

_NC = 2
_NS = 16
_NW = _NC * _NS
_L = 16

_DIM = 32
_NG = 20
_NNEG = 20


def _sc_scores(center, ng2d, ctx, neg2d, W_word, W_ngram, W_ctx, *, B, C):
    per_w = B // _NW
    n_chunks = per_w // C
    nrow = C * _NG
    nblk = nrow // 128
    n_groups = C // _L

    mesh = plsc.VectorSubcoreMesh(
        core_axis_name="c", subcore_axis_name="s",
        num_cores=_NC, num_subcores=_NS)

    @functools.partial(
        pl.kernel,
        out_type=(
            jax.ShapeDtypeStruct((B,), jnp.float32),
            jax.ShapeDtypeStruct((_NW, _NNEG, per_w), jnp.float32),
        ),
        mesh=mesh,
        scratch_types=dict(
            cidx=pltpu.VMEM((C,), jnp.int32),
            pidx=pltpu.VMEM((C,), jnp.int32),
            gidx2=pltpu.VMEM((C, _NG), jnp.int32),
            nidx2=pltpu.VMEM((C, _NNEG), jnp.int32),
            gidx=pltpu.VMEM((nrow,), jnp.int32),
            nidx=pltpu.VMEM((nrow,), jnp.int32),
            wrow=pltpu.VMEM((C, _DIM), jnp.float32),
            prow=pltpu.VMEM((C, _DIM), jnp.float32),
            grow=pltpu.VMEM((nrow, _DIM), jnp.float32),
            nrow_b=pltpu.VMEM((nrow, _DIM), jnp.float32),
            poss=pltpu.VMEM((per_w,), jnp.float32),
            negs=pltpu.VMEM((_NNEG, per_w), jnp.float32),
            sem=pltpu.SemaphoreType.DMA,
        ),
        compiler_params=pltpu.CompilerParams(needs_layout_passes=False, use_tc_tiling_on_sc=False),
    )
    def sck(center_h, ng_h, ctx_h, neg_h, ww_h, wn_h, wc_h, pos_o, neg_o,
            cidx, pidx, gidx2, nidx2, gidx, nidx, wrow, prow, grow, nrow_b,
            poss, negs, sem):
        wid = lax.axis_index("s") * _NC + lax.axis_index("c")

        def xform(r):
            return (((r >> 9) << 9) + ((r & 127) << 2) + ((r >> 7) & 3))

        def chunk_body(g, carry):
            base = wid * per_w + g * C
            pltpu.sync_copy(center_h.at[pl.ds(base, C)], cidx)
            pltpu.sync_copy(ctx_h.at[pl.ds(base, C)], pidx)
            pltpu.sync_copy(ng_h.at[pl.ds(base, C)], gidx2)
            pltpu.sync_copy(neg_h.at[pl.ds(base, C)], nidx2)
            for q in range(C // _L):
                rid = q * _L + lax.iota(jnp.int32, _L)
                cidx[pl.ds(q * _L, _L)] = xform(cidx[pl.ds(q * _L, _L)])
                pidx[pl.ds(q * _L, _L)] = xform(pidx[pl.ds(q * _L, _L)])
                for n in range(_NG):
                    ncol = jnp.full((_L,), n, jnp.int32)
                    gidx[pl.ds(n * C + q * _L, _L)] = xform(plsc.load_gather(
                        gidx2, [rid, ncol]))
                    nidx[pl.ds(n * C + q * _L, _L)] = xform(plsc.load_gather(
                        nidx2, [rid, ncol]))

            cps = [
                pltpu.async_copy(ww_h.at[cidx], wrow, sem),
                pltpu.async_copy(wc_h.at[pidx], prow, sem),
            ]
            for j in range(nblk):
                cps.append(pltpu.async_copy(
                    wn_h.at[gidx.at[pl.ds(j * 128, 128)]],
                    grow.at[pl.ds(j * 128, 128)], sem))
                cps.append(pltpu.async_copy(
                    wc_h.at[nidx.at[pl.ds(j * 128, 128)]],
                    nrow_b.at[pl.ds(j * 128, 128)], sem))
            for cp in cps:
                cp.wait()

            def group_body(sb, carry2):
                rid = sb * _L + lax.iota(jnp.int32, _L)
                cols = [jnp.full((_L,), d, jnp.int32) for d in range(_DIM)]
                cd = [plsc.load_gather(grow, [rid, cols[d]])
                      for d in range(_DIM)]
                for j in range(1, _NG):
                    rgj = rid + j * C
                    for d in range(_DIM):
                        cd[d] = cd[d] + plsc.load_gather(grow, [rgj, cols[d]])
                for d in range(_DIM):
                    cd[d] = (cd[d] * (1.0 / _NG)
                             + plsc.load_gather(wrow, [rid, cols[d]]))
                accs = [jnp.zeros((_L,), jnp.float32) for _ in range(4)]
                for d in range(_DIM):
                    accs[d % 4] = accs[d % 4] + cd[d] * plsc.load_gather(
                        prow, [rid, cols[d]])
                off = g * C + sb * _L
                poss[pl.ds(off, _L)] = (accs[0] + accs[1]) + (accs[2] + accs[3])
                for j in range(_NNEG):
                    rgj = rid + j * C
                    accs = [jnp.zeros((_L,), jnp.float32) for _ in range(4)]
                    for d in range(_DIM):
                        accs[d % 4] = accs[d % 4] + cd[d] * plsc.load_gather(
                            nrow_b, [rgj, cols[d]])
                    negs[j, pl.ds(off, _L)] = ((accs[0] + accs[1])
                                               + (accs[2] + accs[3]))
                return carry2

            lax.fori_loop(0, n_groups, group_body, 0)
            return carry

        lax.fori_loop(0, n_chunks, chunk_body, 0)
        pltpu.sync_copy(poss, pos_o.at[pl.ds(wid * per_w, per_w)])
        pltpu.sync_copy(negs, neg_o.at[wid])

    return sck(center, ng2d, ctx, neg2d, W_word, W_ngram, W_ctx)


def _tc_fmt(wt_w, wt_n, wt_c, *, grid):
    def body(xw_ref, xn_ref, xc_ref, ow_ref, on_ref, oc_ref):
        for x_ref, o_ref in ((xw_ref, ow_ref), (xn_ref, on_ref),
                             (xc_ref, oc_ref)):
            xt = x_ref[...].T
            for q in range(4):
                o_ref[:, q * 32:(q + 1) * 32] = xt[q * 128:(q + 1) * 128, :]

    out = jax.ShapeDtypeStruct((grid * 128, 128), jnp.float32)
    in_spec = pl.BlockSpec((_DIM, 512), lambda i: (0, i))
    out_spec = pl.BlockSpec((128, 128), lambda i: (i, 0))
    return pl.pallas_call(
        body,
        grid=(grid,),
        in_specs=[in_spec] * 3,
        out_specs=[out_spec] * 3,
        out_shape=(out, out, out),
    )(wt_w, wt_n, wt_c)


def _tc_loss(pos2d, neg2d, *, B):
    def body(pos_ref, neg_ref, o_ref):
        p = pos_ref[...]
        n = neg_ref[...]
        pls = -jnp.log(jax.nn.sigmoid(p) + 1e-10)
        nls = -jnp.log(jax.nn.sigmoid(-n) + 1e-10)
        o_ref[0, 0] = (jnp.sum(pls) + jnp.sum(nls)) * (1.0 / B)

    return pl.pallas_call(
        body,
        out_shape=jax.ShapeDtypeStruct((1, 1), jnp.float32),
        out_specs=pl.BlockSpec(memory_space=pltpu.SMEM),
    )(pos2d, neg2d)


def kernel(center_word, ngrams, context_words, neg_words, W_word, W_ngram, W_ctx):
    B = center_word.shape[0]
    C = 64
    center = center_word.astype(jnp.int32)
    ctx = context_words.astype(jnp.int32)
    ng2d = ngrams.astype(jnp.int32)
    neg2d = neg_words.astype(jnp.int32)
    V = W_word.shape[0]
    grid = (V + 511) // 512
    VP = grid * 512
    ww_l, wn_l, wc_l = _tc_fmt(W_word.T, W_ngram.T, W_ctx.T, grid=grid)
    pos_s, neg_s = _sc_scores(center, ng2d, ctx, neg2d,
                              ww_l.reshape(VP, _DIM), wn_l.reshape(VP, _DIM),
                              wc_l.reshape(VP, _DIM), B=B, C=C)
    out = _tc_loss(pos_s.reshape(128, -1), neg_s.reshape(-1, 128), B=B)
    return out.reshape(())


# speedup vs baseline: 1.5766x; 1.5766x over previous
import functools

import jax
import jax.numpy as jnp
from jax import lax
from jax.experimental import pallas as pl
from jax.experimental.pallas import tpu as pltpu
from jax.experimental.pallas import tpu_sc as plsc

_NC = 2
_NS = 16
_NW = _NC * _NS
_L = 16

_DIM = 32
_NG = 20
_NNEG = 20


def _sc_scores(center, ng2d, ctx, neg2d, W_word, W_ngram, W_ctx, *, B, C):
    per_w = B // _NW
    n_chunks = per_w // C
    nrow = C * _NG
    nblk = nrow // 128
    n_groups = C // _L

    mesh = plsc.VectorSubcoreMesh(
        core_axis_name="c", subcore_axis_name="s",
        num_cores=_NC, num_subcores=_NS)

    @functools.partial(
        pl.kernel,
        out_type=(
            jax.ShapeDtypeStruct((B,), jnp.float32),
            jax.ShapeDtypeStruct((_NW, _NNEG, per_w), jnp.float32),
        ),
        mesh=mesh,
        scratch_types=dict(
            cidx=pltpu.VMEM((C,), jnp.int32),
            pidx=pltpu.VMEM((C,), jnp.int32),
            gidx2=pltpu.VMEM((C, _NG), jnp.int32),
            nidx2=pltpu.VMEM((C, _NNEG), jnp.int32),
            gidx=pltpu.VMEM((nrow,), jnp.int32),
            nidx=pltpu.VMEM((nrow,), jnp.int32),
            wrow=pltpu.VMEM((C, _DIM), jnp.float32),
            prow=pltpu.VMEM((C, _DIM), jnp.float32),
            grow=pltpu.VMEM((nrow, _DIM), jnp.float32),
            nrow_b=pltpu.VMEM((nrow, _DIM), jnp.float32),
            poss=pltpu.VMEM((per_w,), jnp.float32),
            negs=pltpu.VMEM((_NNEG, per_w), jnp.float32),
            sem=pltpu.SemaphoreType.DMA,
        ),
        compiler_params=pltpu.CompilerParams(needs_layout_passes=False, use_tc_tiling_on_sc=False),
    )
    def sck(center_h, ng_h, ctx_h, neg_h, ww_h, wn_h, wc_h, pos_o, neg_o,
            cidx, pidx, gidx2, nidx2, gidx, nidx, wrow, prow, grow, nrow_b,
            poss, negs, sem):
        wid = lax.axis_index("s") * _NC + lax.axis_index("c")

        def xform(r):
            return (((r >> 11) << 11) + ((r & 511) << 2) + ((r >> 9) & 3))

        def chunk_body(g, carry):
            base = wid * per_w + g * C
            pltpu.sync_copy(center_h.at[pl.ds(base, C)], cidx)
            pltpu.sync_copy(ctx_h.at[pl.ds(base, C)], pidx)
            pltpu.sync_copy(ng_h.at[pl.ds(base, C)], gidx2)
            pltpu.sync_copy(neg_h.at[pl.ds(base, C)], nidx2)
            for q in range(C // _L):
                rid = q * _L + lax.iota(jnp.int32, _L)
                cidx[pl.ds(q * _L, _L)] = xform(cidx[pl.ds(q * _L, _L)])
                pidx[pl.ds(q * _L, _L)] = xform(pidx[pl.ds(q * _L, _L)])
                for n in range(_NG):
                    ncol = jnp.full((_L,), n, jnp.int32)
                    gidx[pl.ds(n * C + q * _L, _L)] = xform(plsc.load_gather(
                        gidx2, [rid, ncol]))
                    nidx[pl.ds(n * C + q * _L, _L)] = xform(plsc.load_gather(
                        nidx2, [rid, ncol]))

            cps = [
                pltpu.async_copy(ww_h.at[cidx], wrow, sem),
                pltpu.async_copy(wc_h.at[pidx], prow, sem),
            ]
            for j in range(nblk):
                cps.append(pltpu.async_copy(
                    wn_h.at[gidx.at[pl.ds(j * 128, 128)]],
                    grow.at[pl.ds(j * 128, 128)], sem))
                cps.append(pltpu.async_copy(
                    wc_h.at[nidx.at[pl.ds(j * 128, 128)]],
                    nrow_b.at[pl.ds(j * 128, 128)], sem))
            for cp in cps:
                cp.wait()

            def group_body(sb, carry2):
                rid = sb * _L + lax.iota(jnp.int32, _L)
                cols = [jnp.full((_L,), d, jnp.int32) for d in range(_DIM)]
                cd = [plsc.load_gather(grow, [rid, cols[d]])
                      for d in range(_DIM)]
                for j in range(1, _NG):
                    rgj = rid + j * C
                    for d in range(_DIM):
                        cd[d] = cd[d] + plsc.load_gather(grow, [rgj, cols[d]])
                for d in range(_DIM):
                    cd[d] = (cd[d] * (1.0 / _NG)
                             + plsc.load_gather(wrow, [rid, cols[d]]))
                accs = [jnp.zeros((_L,), jnp.float32) for _ in range(4)]
                for d in range(_DIM):
                    accs[d % 4] = accs[d % 4] + cd[d] * plsc.load_gather(
                        prow, [rid, cols[d]])
                off = g * C + sb * _L
                poss[pl.ds(off, _L)] = (accs[0] + accs[1]) + (accs[2] + accs[3])
                for j in range(_NNEG):
                    rgj = rid + j * C
                    accs = [jnp.zeros((_L,), jnp.float32) for _ in range(4)]
                    for d in range(_DIM):
                        accs[d % 4] = accs[d % 4] + cd[d] * plsc.load_gather(
                            nrow_b, [rgj, cols[d]])
                    negs[j, pl.ds(off, _L)] = ((accs[0] + accs[1])
                                               + (accs[2] + accs[3]))
                return carry2

            lax.fori_loop(0, n_groups, group_body, 0)
            return carry

        lax.fori_loop(0, n_chunks, chunk_body, 0)
        pltpu.sync_copy(poss, pos_o.at[pl.ds(wid * per_w, per_w)])
        pltpu.sync_copy(negs, neg_o.at[wid])

    return sck(center, ng2d, ctx, neg2d, W_word, W_ngram, W_ctx)


def _tc_fmt(wt_w, wt_n, wt_c, *, grid):
    def body(xw_ref, xn_ref, xc_ref, ow_ref, on_ref, oc_ref):
        for x_ref, o_ref in ((xw_ref, ow_ref), (xn_ref, on_ref),
                             (xc_ref, oc_ref)):
            xt = x_ref[...].T
            for q in range(4):
                o_ref[:, q * 32:(q + 1) * 32] = xt[q * 512:(q + 1) * 512, :]

    out = jax.ShapeDtypeStruct((grid * 512, 128), jnp.float32)
    in_spec = pl.BlockSpec((_DIM, 2048), lambda i: (0, i))
    out_spec = pl.BlockSpec((512, 128), lambda i: (i, 0))
    return pl.pallas_call(
        body,
        grid=(grid,),
        in_specs=[in_spec] * 3,
        out_specs=[out_spec] * 3,
        out_shape=(out, out, out),
    )(wt_w, wt_n, wt_c)


def _tc_loss(pos2d, neg2d, *, B):
    def body(pos_ref, neg_ref, o_ref):
        p = pos_ref[...]
        n = neg_ref[...]
        pls = -jnp.log(jax.nn.sigmoid(p) + 1e-10)
        nls = -jnp.log(jax.nn.sigmoid(-n) + 1e-10)
        o_ref[0, 0] = (jnp.sum(pls) + jnp.sum(nls)) * (1.0 / B)

    return pl.pallas_call(
        body,
        out_shape=jax.ShapeDtypeStruct((1, 1), jnp.float32),
        out_specs=pl.BlockSpec(memory_space=pltpu.SMEM),
    )(pos2d, neg2d)


def kernel(center_word, ngrams, context_words, neg_words, W_word, W_ngram, W_ctx):
    B = center_word.shape[0]
    C = 64
    center = center_word.astype(jnp.int32)
    ctx = context_words.astype(jnp.int32)
    ng2d = ngrams.astype(jnp.int32)
    neg2d = neg_words.astype(jnp.int32)
    V = W_word.shape[0]
    grid = (V + 2047) // 2048
    VP = grid * 2048
    ww_l, wn_l, wc_l = _tc_fmt(W_word.T, W_ngram.T, W_ctx.T, grid=grid)
    pos_s, neg_s = _sc_scores(center, ng2d, ctx, neg2d,
                              ww_l.reshape(VP, _DIM), wn_l.reshape(VP, _DIM),
                              wc_l.reshape(VP, _DIM), B=B, C=C)
    out = _tc_loss(pos_s.reshape(128, -1), neg_s.reshape(-1, 128), B=B)
    return out.reshape(())


# speedup vs baseline: 1.6367x; 1.0381x over previous
import functools

import jax
import jax.numpy as jnp
from jax import lax
from jax.experimental import pallas as pl
from jax.experimental.pallas import tpu as pltpu
from jax.experimental.pallas import tpu_sc as plsc

_NC = 2
_NS = 16
_NW = _NC * _NS
_L = 16

_DIM = 32
_NG = 20
_NNEG = 20


def _sc_scores(center, ng2d, ctx, neg2d, W_word, W_ngram, W_ctx, *, B, C):
    per_w = B // _NW
    n_chunks = per_w // C
    nrow = C * _NG
    nblk = nrow // 128
    n_groups = C // _L

    mesh = plsc.VectorSubcoreMesh(
        core_axis_name="c", subcore_axis_name="s",
        num_cores=_NC, num_subcores=_NS)

    @functools.partial(
        pl.kernel,
        out_type=(
            jax.ShapeDtypeStruct((B,), jnp.float32),
            jax.ShapeDtypeStruct((_NW, _NNEG, per_w), jnp.float32),
        ),
        mesh=mesh,
        scratch_types=dict(
            cidx=pltpu.VMEM((C,), jnp.int32),
            pidx=pltpu.VMEM((C,), jnp.int32),
            gidx2=pltpu.VMEM((C, _NG), jnp.int32),
            nidx2=pltpu.VMEM((C, _NNEG), jnp.int32),
            gidx=pltpu.VMEM((nrow,), jnp.int32),
            nidx=pltpu.VMEM((nrow,), jnp.int32),
            wrow=pltpu.VMEM((C, _DIM), jnp.float32),
            prow=pltpu.VMEM((C, _DIM), jnp.float32),
            grow=pltpu.VMEM((nrow, _DIM), jnp.float32),
            nrow_b=pltpu.VMEM((nrow, _DIM), jnp.float32),
            poss=pltpu.VMEM((per_w,), jnp.float32),
            negs=pltpu.VMEM((_NNEG, per_w), jnp.float32),
            sem=pltpu.SemaphoreType.DMA,
        ),
        compiler_params=pltpu.CompilerParams(needs_layout_passes=False, use_tc_tiling_on_sc=False),
    )
    def sck(center_h, ng_h, ctx_h, neg_h, ww_h, wn_h, wc_h, pos_o, neg_o,
            cidx, pidx, gidx2, nidx2, gidx, nidx, wrow, prow, grow, nrow_b,
            poss, negs, sem):
        wid = lax.axis_index("s") * _NC + lax.axis_index("c")

        def xform(r):
            return (((r >> 13) << 13) + ((r & 2047) << 2) + ((r >> 11) & 3))

        def chunk_body(g, carry):
            base = wid * per_w + g * C
            pltpu.sync_copy(center_h.at[pl.ds(base, C)], cidx)
            pltpu.sync_copy(ctx_h.at[pl.ds(base, C)], pidx)
            pltpu.sync_copy(ng_h.at[pl.ds(base, C)], gidx2)
            pltpu.sync_copy(neg_h.at[pl.ds(base, C)], nidx2)
            for q in range(C // _L):
                rid = q * _L + lax.iota(jnp.int32, _L)
                cidx[pl.ds(q * _L, _L)] = xform(cidx[pl.ds(q * _L, _L)])
                pidx[pl.ds(q * _L, _L)] = xform(pidx[pl.ds(q * _L, _L)])
                for n in range(_NG):
                    ncol = jnp.full((_L,), n, jnp.int32)
                    gidx[pl.ds(n * C + q * _L, _L)] = xform(plsc.load_gather(
                        gidx2, [rid, ncol]))
                    nidx[pl.ds(n * C + q * _L, _L)] = xform(plsc.load_gather(
                        nidx2, [rid, ncol]))

            cps = [
                pltpu.async_copy(ww_h.at[cidx], wrow, sem),
                pltpu.async_copy(wc_h.at[pidx], prow, sem),
            ]
            for j in range(nblk):
                cps.append(pltpu.async_copy(
                    wn_h.at[gidx.at[pl.ds(j * 128, 128)]],
                    grow.at[pl.ds(j * 128, 128)], sem))
                cps.append(pltpu.async_copy(
                    wc_h.at[nidx.at[pl.ds(j * 128, 128)]],
                    nrow_b.at[pl.ds(j * 128, 128)], sem))
            for cp in cps:
                cp.wait()

            def group_body(sb, carry2):
                rid = sb * _L + lax.iota(jnp.int32, _L)
                cols = [jnp.full((_L,), d, jnp.int32) for d in range(_DIM)]
                cd = [plsc.load_gather(grow, [rid, cols[d]])
                      for d in range(_DIM)]
                for j in range(1, _NG):
                    rgj = rid + j * C
                    for d in range(_DIM):
                        cd[d] = cd[d] + plsc.load_gather(grow, [rgj, cols[d]])
                for d in range(_DIM):
                    cd[d] = (cd[d] * (1.0 / _NG)
                             + plsc.load_gather(wrow, [rid, cols[d]]))
                accs = [jnp.zeros((_L,), jnp.float32) for _ in range(4)]
                for d in range(_DIM):
                    accs[d % 4] = accs[d % 4] + cd[d] * plsc.load_gather(
                        prow, [rid, cols[d]])
                off = g * C + sb * _L
                poss[pl.ds(off, _L)] = (accs[0] + accs[1]) + (accs[2] + accs[3])
                for j in range(_NNEG):
                    rgj = rid + j * C
                    accs = [jnp.zeros((_L,), jnp.float32) for _ in range(4)]
                    for d in range(_DIM):
                        accs[d % 4] = accs[d % 4] + cd[d] * plsc.load_gather(
                            nrow_b, [rgj, cols[d]])
                    negs[j, pl.ds(off, _L)] = ((accs[0] + accs[1])
                                               + (accs[2] + accs[3]))
                return carry2

            lax.fori_loop(0, n_groups, group_body, 0)
            return carry

        lax.fori_loop(0, n_chunks, chunk_body, 0)
        pltpu.sync_copy(poss, pos_o.at[pl.ds(wid * per_w, per_w)])
        pltpu.sync_copy(negs, neg_o.at[wid])

    return sck(center, ng2d, ctx, neg2d, W_word, W_ngram, W_ctx)


def _tc_fmt(wt_w, wt_n, wt_c, *, grid):
    def body(xw_ref, xn_ref, xc_ref, ow_ref, on_ref, oc_ref):
        for x_ref, o_ref in ((xw_ref, ow_ref), (xn_ref, on_ref),
                             (xc_ref, oc_ref)):
            xt = x_ref[...].T
            for q in range(4):
                o_ref[:, q * 32:(q + 1) * 32] = xt[q * 2048:(q + 1) * 2048, :]

    out = jax.ShapeDtypeStruct((grid * 2048, 128), jnp.float32)
    in_spec = pl.BlockSpec((_DIM, 8192), lambda i: (0, i))
    out_spec = pl.BlockSpec((2048, 128), lambda i: (i, 0))
    return pl.pallas_call(
        body,
        grid=(grid,),
        in_specs=[in_spec] * 3,
        out_specs=[out_spec] * 3,
        out_shape=(out, out, out),
    )(wt_w, wt_n, wt_c)


def _tc_loss(pos2d, neg2d, *, B):
    def body(pos_ref, neg_ref, o_ref):
        p = pos_ref[...]
        n = neg_ref[...]
        pls = -jnp.log(jax.nn.sigmoid(p) + 1e-10)
        nls = -jnp.log(jax.nn.sigmoid(-n) + 1e-10)
        o_ref[0, 0] = (jnp.sum(pls) + jnp.sum(nls)) * (1.0 / B)

    return pl.pallas_call(
        body,
        out_shape=jax.ShapeDtypeStruct((1, 1), jnp.float32),
        out_specs=pl.BlockSpec(memory_space=pltpu.SMEM),
    )(pos2d, neg2d)


def kernel(center_word, ngrams, context_words, neg_words, W_word, W_ngram, W_ctx):
    B = center_word.shape[0]
    C = 64
    center = center_word.astype(jnp.int32)
    ctx = context_words.astype(jnp.int32)
    ng2d = ngrams.astype(jnp.int32)
    neg2d = neg_words.astype(jnp.int32)
    V = W_word.shape[0]
    grid = (V + 8191) // 8192
    VP = grid * 8192
    ww_l, wn_l, wc_l = _tc_fmt(W_word.T, W_ngram.T, W_ctx.T, grid=grid)
    pos_s, neg_s = _sc_scores(center, ng2d, ctx, neg2d,
                              ww_l.reshape(VP, _DIM), wn_l.reshape(VP, _DIM),
                              wc_l.reshape(VP, _DIM), B=B, C=C)
    out = _tc_loss(pos_s.reshape(128, -1), neg_s.reshape(-1, 128), B=B)
    return out.reshape(())


# speedup vs baseline: 1.9306x; 1.1796x over previous
import functools

import jax
import jax.numpy as jnp
from jax import lax
from jax.experimental import pallas as pl
from jax.experimental.pallas import tpu as pltpu
from jax.experimental.pallas import tpu_sc as plsc

_NC = 2
_NS = 16
_NW = _NC * _NS
_L = 16

_DIM = 32
_NG = 20
_NNEG = 20


def _sc_scores(center, ng2d, ctx, neg2d, W_word, W_ngram, W_ctx, *, B, C):
    per_w = B // _NW
    n_chunks = per_w // C
    nrow = C * _NG
    nblk = nrow // 128
    n_groups = C // _L

    mesh = plsc.VectorSubcoreMesh(
        core_axis_name="c", subcore_axis_name="s",
        num_cores=_NC, num_subcores=_NS)

    @functools.partial(
        pl.kernel,
        out_type=(
            jax.ShapeDtypeStruct((B,), jnp.float32),
            jax.ShapeDtypeStruct((_NW, _NNEG, per_w), jnp.float32),
        ),
        mesh=mesh,
        scratch_types=dict(
            cidx=pltpu.VMEM((C,), jnp.int32),
            pidx=pltpu.VMEM((C,), jnp.int32),
            gidx2=pltpu.VMEM((C, _NG), jnp.int32),
            nidx2=pltpu.VMEM((C, _NNEG), jnp.int32),
            gidx=pltpu.VMEM((nrow,), jnp.int32),
            nidx=pltpu.VMEM((nrow,), jnp.int32),
            wrow=pltpu.VMEM((C, _DIM), jnp.float32),
            prow=pltpu.VMEM((C, _DIM), jnp.float32),
            cacc=pltpu.VMEM((C, _DIM), jnp.float32),
            nrow_b=pltpu.VMEM((nrow, _DIM), jnp.float32),
            poss=pltpu.VMEM((per_w,), jnp.float32),
            negs=pltpu.VMEM((_NNEG, per_w), jnp.float32),
            sem=pltpu.SemaphoreType.DMA,
        ),
        compiler_params=pltpu.CompilerParams(needs_layout_passes=False, use_tc_tiling_on_sc=False),
    )
    def sck(center_h, ng_h, ctx_h, neg_h, ww_h, wn_h, wc_h, pos_o, neg_o,
            cidx, pidx, gidx2, nidx2, gidx, nidx, wrow, prow, cacc, nrow_b,
            poss, negs, sem):
        wid = lax.axis_index("s") * _NC + lax.axis_index("c")

        def xform(r):
            return (((r >> 13) << 13) + ((r & 2047) << 2) + ((r >> 11) & 3))

        def chunk_body(g, carry):
            base = wid * per_w + g * C
            pltpu.sync_copy(center_h.at[pl.ds(base, C)], cidx)
            pltpu.sync_copy(ctx_h.at[pl.ds(base, C)], pidx)
            pltpu.sync_copy(ng_h.at[pl.ds(base, C)], gidx2)
            pltpu.sync_copy(neg_h.at[pl.ds(base, C)], nidx2)
            for q in range(C // _L):
                rid = q * _L + lax.iota(jnp.int32, _L)
                cidx[pl.ds(q * _L, _L)] = xform(cidx[pl.ds(q * _L, _L)])
                pidx[pl.ds(q * _L, _L)] = xform(pidx[pl.ds(q * _L, _L)])
                for n in range(_NG):
                    ncol = jnp.full((_L,), n, jnp.int32)
                    gidx[pl.ds(n * C + q * _L, _L)] = xform(plsc.load_gather(
                        gidx2, [rid, ncol]))
                    nidx[pl.ds(n * C + q * _L, _L)] = xform(plsc.load_gather(
                        nidx2, [rid, ncol]))

            zero = jnp.zeros((_L,), jnp.float32)
            for i in range(C):
                cacc[i, pl.ds(0, _L)] = zero
                cacc[i, pl.ds(_L, _L)] = zero
            cps = [
                pltpu.async_copy(ww_h.at[cidx], wrow, sem),
                pltpu.async_copy(wc_h.at[pidx], prow, sem),
            ]
            for n in range(_NG):
                cps.append(pltpu.async_copy(
                    wn_h.at[gidx.at[pl.ds(n * C, C)]], cacc, sem, add=True))
            for j in range(nblk):
                cps.append(pltpu.async_copy(
                    wc_h.at[nidx.at[pl.ds(j * 128, 128)]],
                    nrow_b.at[pl.ds(j * 128, 128)], sem))
            for cp in cps:
                cp.wait()

            def group_body(sb, carry2):
                rid = sb * _L + lax.iota(jnp.int32, _L)
                cols = [jnp.full((_L,), d, jnp.int32) for d in range(_DIM)]
                cd = [plsc.load_gather(cacc, [rid, cols[d]]) * (1.0 / _NG)
                      + plsc.load_gather(wrow, [rid, cols[d]])
                      for d in range(_DIM)]
                accs = [jnp.zeros((_L,), jnp.float32) for _ in range(4)]
                for d in range(_DIM):
                    accs[d % 4] = accs[d % 4] + cd[d] * plsc.load_gather(
                        prow, [rid, cols[d]])
                off = g * C + sb * _L
                poss[pl.ds(off, _L)] = (accs[0] + accs[1]) + (accs[2] + accs[3])
                for j in range(_NNEG):
                    rgj = rid + j * C
                    accs = [jnp.zeros((_L,), jnp.float32) for _ in range(4)]
                    for d in range(_DIM):
                        accs[d % 4] = accs[d % 4] + cd[d] * plsc.load_gather(
                            nrow_b, [rgj, cols[d]])
                    negs[j, pl.ds(off, _L)] = ((accs[0] + accs[1])
                                               + (accs[2] + accs[3]))
                return carry2

            lax.fori_loop(0, n_groups, group_body, 0)
            return carry

        lax.fori_loop(0, n_chunks, chunk_body, 0)
        pltpu.sync_copy(poss, pos_o.at[pl.ds(wid * per_w, per_w)])
        pltpu.sync_copy(negs, neg_o.at[wid])

    return sck(center, ng2d, ctx, neg2d, W_word, W_ngram, W_ctx)


def _tc_fmt(wt_w, wt_n, wt_c, *, grid):
    def body(xw_ref, xn_ref, xc_ref, ow_ref, on_ref, oc_ref):
        for x_ref, o_ref in ((xw_ref, ow_ref), (xn_ref, on_ref),
                             (xc_ref, oc_ref)):
            xt = x_ref[...].T
            for q in range(4):
                o_ref[:, q * 32:(q + 1) * 32] = xt[q * 2048:(q + 1) * 2048, :]

    out = jax.ShapeDtypeStruct((grid * 2048, 128), jnp.float32)
    in_spec = pl.BlockSpec((_DIM, 8192), lambda i: (0, i))
    out_spec = pl.BlockSpec((2048, 128), lambda i: (i, 0))
    return pl.pallas_call(
        body,
        grid=(grid,),
        in_specs=[in_spec] * 3,
        out_specs=[out_spec] * 3,
        out_shape=(out, out, out),
    )(wt_w, wt_n, wt_c)


def _tc_loss(pos2d, neg2d, *, B):
    def body(pos_ref, neg_ref, o_ref):
        p = pos_ref[...]
        n = neg_ref[...]
        pls = -jnp.log(jax.nn.sigmoid(p) + 1e-10)
        nls = -jnp.log(jax.nn.sigmoid(-n) + 1e-10)
        o_ref[0, 0] = (jnp.sum(pls) + jnp.sum(nls)) * (1.0 / B)

    return pl.pallas_call(
        body,
        out_shape=jax.ShapeDtypeStruct((1, 1), jnp.float32),
        out_specs=pl.BlockSpec(memory_space=pltpu.SMEM),
    )(pos2d, neg2d)


def kernel(center_word, ngrams, context_words, neg_words, W_word, W_ngram, W_ctx):
    B = center_word.shape[0]
    C = 64
    center = center_word.astype(jnp.int32)
    ctx = context_words.astype(jnp.int32)
    ng2d = ngrams.astype(jnp.int32)
    neg2d = neg_words.astype(jnp.int32)
    V = W_word.shape[0]
    grid = (V + 8191) // 8192
    VP = grid * 8192
    ww_l, wn_l, wc_l = _tc_fmt(W_word.T, W_ngram.T, W_ctx.T, grid=grid)
    pos_s, neg_s = _sc_scores(center, ng2d, ctx, neg2d,
                              ww_l.reshape(VP, _DIM), wn_l.reshape(VP, _DIM),
                              wc_l.reshape(VP, _DIM), B=B, C=C)
    out = _tc_loss(pos_s.reshape(128, -1), neg_s.reshape(-1, 128), B=B)
    return out.reshape(())


# speedup vs baseline: 1.9316x; 1.0005x over previous
import functools

import jax
import jax.numpy as jnp
from jax import lax
from jax.experimental import pallas as pl
from jax.experimental.pallas import tpu as pltpu
from jax.experimental.pallas import tpu_sc as plsc

_NC = 2
_NS = 16
_NW = _NC * _NS
_L = 16

_DIM = 32
_NG = 20
_NNEG = 20


def _sc_scores(center, ng2d, ctx, neg2d, W_word, W_ngram, W_ctx, *, B, C):
    per_w = B // _NW
    n_chunks = per_w // C
    nrow = C * _NG
    nblk = nrow // 128
    n_groups = C // _L

    mesh = plsc.VectorSubcoreMesh(
        core_axis_name="c", subcore_axis_name="s",
        num_cores=_NC, num_subcores=_NS)

    @functools.partial(
        pl.kernel,
        out_type=(
            jax.ShapeDtypeStruct((B,), jnp.float32),
            jax.ShapeDtypeStruct((_NW, _NNEG, per_w), jnp.float32),
        ),
        mesh=mesh,
        scratch_types=dict(
            cidx=pltpu.VMEM((C,), jnp.int32),
            pidx=pltpu.VMEM((C,), jnp.int32),
            gidx2=pltpu.VMEM((C, _NG), jnp.int32),
            nidx2=pltpu.VMEM((C, _NNEG), jnp.int32),
            gidx=pltpu.VMEM((nrow,), jnp.int32),
            nidx=pltpu.VMEM((nrow,), jnp.int32),
            wrow=pltpu.VMEM((C, _DIM), jnp.float32),
            prow=pltpu.VMEM((C, _DIM), jnp.float32),
            cacc=pltpu.VMEM((C, _DIM), jnp.float32),
            nrow_b=pltpu.VMEM((nrow, _DIM), jnp.float32),
            poss=pltpu.VMEM((per_w,), jnp.float32),
            negs=pltpu.VMEM((_NNEG, per_w), jnp.float32),
            sem=pltpu.SemaphoreType.DMA,
        ),
        compiler_params=pltpu.CompilerParams(needs_layout_passes=False, use_tc_tiling_on_sc=False),
    )
    def sck(center_h, ng_h, ctx_h, neg_h, ww_h, wn_h, wc_h, pos_o, neg_o,
            cidx, pidx, gidx2, nidx2, gidx, nidx, wrow, prow, cacc, nrow_b,
            poss, negs, sem):
        wid = lax.axis_index("s") * _NC + lax.axis_index("c")

        def xform(r):
            return (((r >> 14) << 14) + ((r & 4095) << 2) + ((r >> 12) & 3))

        def chunk_body(g, carry):
            base = wid * per_w + g * C
            pltpu.sync_copy(center_h.at[pl.ds(base, C)], cidx)
            pltpu.sync_copy(ctx_h.at[pl.ds(base, C)], pidx)
            pltpu.sync_copy(ng_h.at[pl.ds(base, C)], gidx2)
            pltpu.sync_copy(neg_h.at[pl.ds(base, C)], nidx2)
            for q in range(C // _L):
                rid = q * _L + lax.iota(jnp.int32, _L)
                cidx[pl.ds(q * _L, _L)] = xform(cidx[pl.ds(q * _L, _L)])
                pidx[pl.ds(q * _L, _L)] = xform(pidx[pl.ds(q * _L, _L)])
                for n in range(_NG):
                    ncol = jnp.full((_L,), n, jnp.int32)
                    gidx[pl.ds(n * C + q * _L, _L)] = xform(plsc.load_gather(
                        gidx2, [rid, ncol]))
                    nidx[pl.ds(n * C + q * _L, _L)] = xform(plsc.load_gather(
                        nidx2, [rid, ncol]))

            zero = jnp.zeros((_L,), jnp.float32)
            for i in range(C):
                cacc[i, pl.ds(0, _L)] = zero
                cacc[i, pl.ds(_L, _L)] = zero
            cps = [
                pltpu.async_copy(ww_h.at[cidx], wrow, sem),
                pltpu.async_copy(wc_h.at[pidx], prow, sem),
            ]
            for n in range(_NG):
                cps.append(pltpu.async_copy(
                    wn_h.at[gidx.at[pl.ds(n * C, C)]], cacc, sem, add=True))
            for j in range(nblk):
                cps.append(pltpu.async_copy(
                    wc_h.at[nidx.at[pl.ds(j * 128, 128)]],
                    nrow_b.at[pl.ds(j * 128, 128)], sem))
            for cp in cps:
                cp.wait()

            def group_body(sb, carry2):
                rid = sb * _L + lax.iota(jnp.int32, _L)
                cols = [jnp.full((_L,), d, jnp.int32) for d in range(_DIM)]
                cd = [plsc.load_gather(cacc, [rid, cols[d]]) * (1.0 / _NG)
                      + plsc.load_gather(wrow, [rid, cols[d]])
                      for d in range(_DIM)]
                accs = [jnp.zeros((_L,), jnp.float32) for _ in range(4)]
                for d in range(_DIM):
                    accs[d % 4] = accs[d % 4] + cd[d] * plsc.load_gather(
                        prow, [rid, cols[d]])
                off = g * C + sb * _L
                poss[pl.ds(off, _L)] = (accs[0] + accs[1]) + (accs[2] + accs[3])
                for j in range(_NNEG):
                    rgj = rid + j * C
                    accs = [jnp.zeros((_L,), jnp.float32) for _ in range(4)]
                    for d in range(_DIM):
                        accs[d % 4] = accs[d % 4] + cd[d] * plsc.load_gather(
                            nrow_b, [rgj, cols[d]])
                    negs[j, pl.ds(off, _L)] = ((accs[0] + accs[1])
                                               + (accs[2] + accs[3]))
                return carry2

            lax.fori_loop(0, n_groups, group_body, 0)
            return carry

        lax.fori_loop(0, n_chunks, chunk_body, 0)
        pltpu.sync_copy(poss, pos_o.at[pl.ds(wid * per_w, per_w)])
        pltpu.sync_copy(negs, neg_o.at[wid])

    return sck(center, ng2d, ctx, neg2d, W_word, W_ngram, W_ctx)


def _tc_fmt(wt_w, wt_n, wt_c, *, grid):
    def body(xw_ref, xn_ref, xc_ref, ow_ref, on_ref, oc_ref):
        for x_ref, o_ref in ((xw_ref, ow_ref), (xn_ref, on_ref),
                             (xc_ref, oc_ref)):
            xt = x_ref[...].T
            for q in range(4):
                o_ref[:, q * 32:(q + 1) * 32] = xt[q * 4096:(q + 1) * 4096, :]

    out = jax.ShapeDtypeStruct((grid * 4096, 128), jnp.float32)
    in_spec = pl.BlockSpec((_DIM, 16384), lambda i: (0, i))
    out_spec = pl.BlockSpec((4096, 128), lambda i: (i, 0))
    return pl.pallas_call(
        body,
        grid=(grid,),
        in_specs=[in_spec] * 3,
        out_specs=[out_spec] * 3,
        out_shape=(out, out, out),
    )(wt_w, wt_n, wt_c)


def _tc_loss(pos2d, neg2d, *, B):
    def body(pos_ref, neg_ref, o_ref):
        p = pos_ref[...]
        n = neg_ref[...]
        pls = -jnp.log(jax.nn.sigmoid(p) + 1e-10)
        nls = -jnp.log(jax.nn.sigmoid(-n) + 1e-10)
        o_ref[0, 0] = (jnp.sum(pls) + jnp.sum(nls)) * (1.0 / B)

    return pl.pallas_call(
        body,
        out_shape=jax.ShapeDtypeStruct((1, 1), jnp.float32),
        out_specs=pl.BlockSpec(memory_space=pltpu.SMEM),
    )(pos2d, neg2d)


def kernel(center_word, ngrams, context_words, neg_words, W_word, W_ngram, W_ctx):
    B = center_word.shape[0]
    C = 64
    center = center_word.astype(jnp.int32)
    ctx = context_words.astype(jnp.int32)
    ng2d = ngrams.astype(jnp.int32)
    neg2d = neg_words.astype(jnp.int32)
    V = W_word.shape[0]
    grid = (V + 16383) // 16384
    VP = grid * 16384
    ww_l, wn_l, wc_l = _tc_fmt(W_word.T, W_ngram.T, W_ctx.T, grid=grid)
    pos_s, neg_s = _sc_scores(center, ng2d, ctx, neg2d,
                              ww_l.reshape(VP, _DIM), wn_l.reshape(VP, _DIM),
                              wc_l.reshape(VP, _DIM), B=B, C=C)
    out = _tc_loss(pos_s.reshape(128, -1), neg_s.reshape(-1, 128), B=B)
    return out.reshape(())
